# baseline (speedup 1.0000x reference)
import jax
import jax.numpy as jnp
from jax import lax
from jax.experimental import pallas as pl
from jax.experimental.pallas import tpu as pltpu
from jax.experimental.pallas import tpu_sc as plsc

B, T, C = 4, 8192, 1024
KK = 8
S = 16
G = T // S
NCB = 8
CB = C // NCB
NLG = CB // 16
NCG = 32
CGS = G // NCG
RB = 256
BT = 2
BS = B - BT
TCHUNK = 512
SLAB = 32
NEG_INF = float("-inf")


def _tc_body(x_ref, gm_ref, xlin_ref):
    tt = pl.program_id(2)
    for cb in range(NCB):
        slab = x_ref[0, :, pl.ds(cb * CB, CB)]

        @pl.when(tt == 0)
        def _():
            gm_ref[0, cb] = slab

        @pl.when(tt != 0)
        def _():
            gm_ref[0, cb] = jnp.maximum(gm_ref[0, cb], slab)
    for r in range(RB):
        xlin_ref[pl.ds(r * C, C)] = x_ref[0, r, :]


def _tc_pass(x):
    return pl.pallas_call(
        _tc_body,
        grid=(BT, 2, S),
        in_specs=[pl.BlockSpec((1, RB, C), lambda b, h, tt: (b, 2 * tt + h, 0))],
        out_specs=[
            pl.BlockSpec((1, NCB, RB, CB), lambda b, h, tt: (b, 0, h, 0)),
            pl.BlockSpec((RB * C,), lambda b, h, tt: (b * 2 * S + tt * 2 + h)),
        ],
        out_shape=[
            jax.ShapeDtypeStruct((BT, NCB, G, CB), jnp.float32),
            jax.ShapeDtypeStruct((BT * T * C,), jnp.float32),
        ],
        compiler_params=pltpu.CompilerParams(
            dimension_semantics=("arbitrary", "arbitrary", "arbitrary")),
    )(x)


def _sc_copy_body(x_hbm, xlin_hbm, gm_hbm, buf0, buf1, gmacc, sem):
    cid = lax.axis_index("c")
    sid = lax.axis_index("s")
    wid = sid * 2 + cid
    bb = wid // 16
    tchunk = lax.rem(wid, 16)
    row0 = tchunk * TCHUNK
    flat0 = (bb * T + row0) * C

    bufs = (buf0, buf1)
    nslab = TCHUNK // SLAB

    def _drain_one_slab(buf):
        pltpu.make_async_copy(
            x_hbm.at[BT, pl.ds(0, SLAB), :], buf, sem).wait()

    for s in range(nslab):
        buf = bufs[s % 2]
        if s >= 2:
            _drain_one_slab(buf)
        pltpu.sync_copy(x_hbm.at[BT + bb, pl.ds(row0 + s * SLAB, SLAB), :],
                        buf)

        def ci_body(ci, _):
            for q in range(SLAB // S):
                def m_body(m, acc):
                    return jnp.maximum(acc, buf[q * S + m, pl.ds(ci * 16, 16)])
                acc = lax.fori_loop(0, S, m_body,
                                    jnp.full((16,), NEG_INF, jnp.float32))
                gmacc[s * (SLAB // S) + q, pl.ds(ci * 16, 16)] = acc
            return _
        lax.fori_loop(0, C // 16, ci_body, 0)

        base = flat0 + s * SLAB * C

        def row_out(r, _):
            pltpu.async_copy(
                buf.at[r], xlin_hbm.at[pl.ds(base + r * C, C)], sem)
            return _
        lax.fori_loop(0, SLAB, row_out, 0)
    _drain_one_slab(buf0)
    _drain_one_slab(buf1)

    ngl = TCHUNK // S
    for cb in range(NCB):
        pltpu.sync_copy(gmacc.at[:, pl.ds(cb * CB, CB)],
                        gm_hbm.at[bb, cb, pl.ds(tchunk * ngl, ngl), :])


def _sc_copy(x):
    mesh = plsc.VectorSubcoreMesh(
        core_axis_name="c", subcore_axis_name="s", num_cores=2,
        num_subcores=16)
    f = pl.kernel(
        _sc_copy_body,
        out_type=(jax.ShapeDtypeStruct((BS * T * C,), jnp.float32),
                  jax.ShapeDtypeStruct((BS, NCB, G, CB), jnp.float32)),
        mesh=mesh,
        scratch_types=[
            pltpu.VMEM((SLAB, C), jnp.float32),
            pltpu.VMEM((SLAB, C), jnp.float32),
            pltpu.VMEM((TCHUNK // S, C), jnp.float32),
            pltpu.SemaphoreType.DMA,
        ],
        compiler_params=pltpu.CompilerParams(needs_layout_passes=False),
    )
    return f(x)


def _insert8(vals, x):
    out = []
    for i in range(KK):
        c = x > vals[i]
        out.append(jnp.where(c, x, vals[i]))
        x = jnp.where(c, vals[i], x)
    return out


def _insert8_kv(vals, idxs, x, g):
    out_v, out_i = [], []
    for i in range(KK):
        c = x > vals[i]
        out_v.append(jnp.where(c, x, vals[i]))
        out_i.append(jnp.where(c, g, idxs[i]))
        x = jnp.where(c, vals[i], x)
        g = jnp.where(c, idxs[i], g)
    return out_v, out_i


def _topk_sc_body(xlina_hbm, xlinb_hbm, gma_hbm, gmb_hbm, out_hbm,
                  gm_v, candv, candg, out_v, *rest):
    idx_vs = rest[:NLG]
    gath_vs = rest[NLG:2 * NLG]
    sem = rest[2 * NLG]
    cid = lax.axis_index("c")
    sid = lax.axis_index("s")
    wid = sid * 2 + cid
    b = wid // NCB
    cb = lax.rem(wid, NCB)
    is_a = b < BT

    @pl.when(is_a)
    def _():
        pltpu.sync_copy(gma_hbm.at[b, cb], gm_v)

    @pl.when(jnp.logical_not(is_a))
    def _():
        pltpu.sync_copy(gmb_hbm.at[b - BT, cb], gm_v)

    lanes = lax.iota(jnp.int32, 16)
    for lg in range(NLG):
        col = lg * 16

        def coarse_body(cg, carry):
            def row_body(j, acc):
                return jnp.maximum(acc, gm_v[cg * CGS + j, pl.ds(col, 16)])
            m = lax.fori_loop(0, CGS, row_body,
                              jnp.full((16,), NEG_INF, jnp.float32))
            return tuple(_insert8(list(carry), m))
        top0 = lax.fori_loop(
            0, NCG, coarse_body,
            tuple(jnp.full((16,), NEG_INF, jnp.float32) for _ in range(KK)))
        tau0 = top0[KK - 1]

        def scan_body(r, cnt):
            x = gm_v[r, pl.ds(col, 16)]
            msk = x >= tau0
            slot = cnt * 16 + lanes
            plsc.store_scatter(candv, [slot], x, mask=msk)
            plsc.store_scatter(candg, [slot],
                               jnp.full((16,), r, jnp.int32), mask=msk)
            return cnt + msk.astype(jnp.int32)
        cnt = lax.fori_loop(0, G, scan_body, jnp.zeros((16,), jnp.int32))
        maxcnt = jnp.max(cnt)

        def ins_body(r, carry):
            vals = list(carry[:KK])
            idxs = list(carry[KK:])
            valid = r < cnt
            x = jnp.where(valid, candv[pl.ds(r * 16, 16)], NEG_INF)
            g = candg[pl.ds(r * 16, 16)]
            vals, idxs = _insert8_kv(vals, idxs, x, g)
            return tuple(vals) + tuple(idxs)
        init = (tuple(jnp.full((16,), NEG_INF, jnp.float32) for _ in range(KK))
                + tuple(jnp.zeros((16,), jnp.int32) for _ in range(KK)))
        res = lax.fori_loop(0, maxcnt, ins_body, init)
        gids = res[KK:]

        cvec = cb * CB + col + lanes

        @pl.when(is_a)
        def _():
            cbase = b * (T * C) + cvec
            for j in range(KK):
                base = gids[j] * C + cbase
                for m in range(S):
                    idx_vs[lg][pl.ds((j * S + m) * 16, 16)] = (
                        base + m * (G * C))

        @pl.when(jnp.logical_not(is_a))
        def _():
            cbase = (b - BT) * (T * C) + cvec
            for j in range(KK):
                base = ((gids[j] >> 5) * (TCHUNK * C)
                        + (gids[j] & 31) * (S * C) + cbase)
                for m in range(S):
                    idx_vs[lg][pl.ds((j * S + m) * 16, 16)] = base + m * C

        @pl.when(is_a)
        def _():
            pltpu.async_copy(xlina_hbm.at[idx_vs[lg]], gath_vs[lg], sem)

        @pl.when(jnp.logical_not(is_a))
        def _():
            pltpu.async_copy(xlinb_hbm.at[idx_vs[lg]], gath_vs[lg], sem)

    for lg in range(NLG):
        pltpu.make_async_copy(
            xlina_hbm.at[idx_vs[lg]], gath_vs[lg], sem).wait()

    for lg in range(NLG):
        col = lg * 16

        def fin_body(q, carry):
            x = gath_vs[lg][pl.ds(q * 16, 16)]
            return tuple(_insert8(list(carry), x))
        top = lax.fori_loop(
            0, KK * S, fin_body,
            tuple(jnp.full((16,), NEG_INF, jnp.float32) for _ in range(KK)))
        for k in range(KK):
            out_v[k, pl.ds(col, 16)] = top[k]

    pltpu.sync_copy(out_v, out_hbm.at[b, cb])


def _topk_sc(xlina, xlinb, gma, gmb):
    mesh = plsc.VectorSubcoreMesh(
        core_axis_name="c", subcore_axis_name="s", num_cores=2,
        num_subcores=16)
    f = pl.kernel(
        _topk_sc_body,
        out_type=jax.ShapeDtypeStruct((B, NCB, KK, CB), jnp.float32),
        mesh=mesh,
        compiler_params=pltpu.CompilerParams(needs_layout_passes=False),
        scratch_types=[
            pltpu.VMEM((G, CB), jnp.float32),
            pltpu.VMEM((G * 16,), jnp.float32),
            pltpu.VMEM((G * 16,), jnp.int32),
            pltpu.VMEM((KK, CB), jnp.float32),
        ] + [pltpu.VMEM((KK * S * 16,), jnp.int32) for _ in range(NLG)]
          + [pltpu.VMEM((KK * S * 16,), jnp.float32) for _ in range(NLG)]
          + [pltpu.SemaphoreType.DMA],
    )
    return f(xlina, xlinb, gma, gmb)


@jax.jit
def kernel(top_k):
    gma, xlina = _tc_pass(top_k)
    xlinb, gmb = _sc_copy(top_k)
    out_blk = _topk_sc(xlina, xlinb, gma, gmb)
    return out_blk.transpose(0, 2, 1, 3).reshape(B, KK, C)


# speedup vs baseline: 1.0828x; 1.0828x over previous
import jax
import jax.numpy as jnp
from jax import lax
from jax.experimental import pallas as pl
from jax.experimental.pallas import tpu as pltpu
from jax.experimental.pallas import tpu_sc as plsc

B, T, C = 4, 8192, 1024
KK = 8
S = 16
G = T // S
NCB = 8
CB = C // NCB
NLG = CB // 16
NCG = 32
CGS = G // NCG
RB = 256
BT = 1
BS = B - BT
NEG_INF = float("-inf")


def _tc_body(x_ref, gm_ref, xlin_ref):
    tt = pl.program_id(2)
    for cb in range(NCB):
        slab = x_ref[0, :, pl.ds(cb * CB, CB)]

        @pl.when(tt == 0)
        def _():
            gm_ref[0, cb] = slab

        @pl.when(tt != 0)
        def _():
            gm_ref[0, cb] = jnp.maximum(gm_ref[0, cb], slab)
    for r in range(RB):
        xlin_ref[pl.ds(r * C, C)] = x_ref[0, r, :]


def _tc_pass(x):
    return pl.pallas_call(
        _tc_body,
        grid=(BT, 2, S),
        in_specs=[pl.BlockSpec((1, RB, C), lambda b, h, tt: (b, 2 * tt + h, 0))],
        out_specs=[
            pl.BlockSpec((1, NCB, RB, CB), lambda b, h, tt: (b, 0, h, 0)),
            pl.BlockSpec((RB * C,), lambda b, h, tt: (b * 2 * S + tt * 2 + h)),
        ],
        out_shape=[
            jax.ShapeDtypeStruct((BT, NCB, G, CB), jnp.float32),
            jax.ShapeDtypeStruct((BT * T * C,), jnp.float32),
        ],
        compiler_params=pltpu.CompilerParams(
            dimension_semantics=("arbitrary", "arbitrary", "arbitrary")),
    )(x)


def _gm_body(x_ref, gm_ref):
    acc = x_ref[0, pl.ds(0, G), :]
    for m in range(1, S):
        acc = jnp.maximum(acc, x_ref[0, pl.ds(m * G, G), :])
    gm_ref[0, 0] = acc


def _tc_gm_rest(x):
    return pl.pallas_call(
        _gm_body,
        grid=(BS, NCB),
        in_specs=[pl.BlockSpec((1, T, CB), lambda b, cb: (b + BT, 0, cb))],
        out_specs=pl.BlockSpec((1, 1, G, CB), lambda b, cb: (b, cb, 0, 0)),
        out_shape=jax.ShapeDtypeStruct((BS, NCB, G, CB), jnp.float32),
    )(x)


def _insert8(vals, x):
    out = []
    for i in range(KK):
        c = x > vals[i]
        out.append(jnp.where(c, x, vals[i]))
        x = jnp.where(c, vals[i], x)
    return out


def _insert8_kv(vals, idxs, x, g):
    out_v, out_i = [], []
    for i in range(KK):
        c = x > vals[i]
        out_v.append(jnp.where(c, x, vals[i]))
        out_i.append(jnp.where(c, g, idxs[i]))
        x = jnp.where(c, vals[i], x)
        g = jnp.where(c, idxs[i], g)
    return out_v, out_i


def _topk_sc_body(xlina_hbm, xlinb_hbm, gma_hbm, gmb_hbm, out_hbm,
                  gm_v, candv, candg, out_v, *rest):
    idx_vs = rest[:NLG]
    gath_vs = rest[NLG:2 * NLG]
    sem = rest[2 * NLG]
    cid = lax.axis_index("c")
    sid = lax.axis_index("s")
    wid = sid * 2 + cid
    b = wid // NCB
    cb = lax.rem(wid, NCB)
    is_a = b < BT

    @pl.when(is_a)
    def _():
        pltpu.sync_copy(gma_hbm.at[b, cb], gm_v)

    @pl.when(jnp.logical_not(is_a))
    def _():
        pltpu.sync_copy(gmb_hbm.at[b - BT, cb], gm_v)

    lanes = lax.iota(jnp.int32, 16)
    for lg in range(NLG):
        col = lg * 16

        def coarse_body(cg, carry):
            def row_body(j, acc):
                return jnp.maximum(acc, gm_v[cg * CGS + j, pl.ds(col, 16)])
            m = lax.fori_loop(0, CGS, row_body,
                              jnp.full((16,), NEG_INF, jnp.float32))
            return tuple(_insert8(list(carry), m))
        top0 = lax.fori_loop(
            0, NCG, coarse_body,
            tuple(jnp.full((16,), NEG_INF, jnp.float32) for _ in range(KK)))
        tau0 = top0[KK - 1]

        def scan_body(r, cnt):
            x = gm_v[r, pl.ds(col, 16)]
            msk = x >= tau0
            slot = cnt * 16 + lanes
            plsc.store_scatter(candv, [slot], x, mask=msk)
            plsc.store_scatter(candg, [slot],
                               jnp.full((16,), r, jnp.int32), mask=msk)
            return cnt + msk.astype(jnp.int32)
        cnt = lax.fori_loop(0, G, scan_body, jnp.zeros((16,), jnp.int32))
        maxcnt = jnp.max(cnt)

        def ins_body(r, carry):
            vals = list(carry[:KK])
            idxs = list(carry[KK:])
            valid = r < cnt
            x = jnp.where(valid, candv[pl.ds(r * 16, 16)], NEG_INF)
            g = candg[pl.ds(r * 16, 16)]
            vals, idxs = _insert8_kv(vals, idxs, x, g)
            return tuple(vals) + tuple(idxs)
        init = (tuple(jnp.full((16,), NEG_INF, jnp.float32) for _ in range(KK))
                + tuple(jnp.zeros((16,), jnp.int32) for _ in range(KK)))
        res = lax.fori_loop(0, maxcnt, ins_body, init)
        gids = res[KK:]

        bq = jnp.where(is_a, b, b - BT)
        cbase = bq * (T * C) + cb * CB + col + lanes
        for j in range(KK):
            base = gids[j] * C + cbase
            for m in range(S):
                idx_vs[lg][pl.ds((j * S + m) * 16, 16)] = base + m * (G * C)

        @pl.when(is_a)
        def _():
            pltpu.async_copy(xlina_hbm.at[idx_vs[lg]], gath_vs[lg], sem)

        @pl.when(jnp.logical_not(is_a))
        def _():
            pltpu.async_copy(xlinb_hbm.at[idx_vs[lg]], gath_vs[lg], sem)

    for lg in range(NLG):
        pltpu.make_async_copy(
            xlina_hbm.at[idx_vs[lg]], gath_vs[lg], sem).wait()

    for lg in range(NLG):
        col = lg * 16

        def fin_body(q, carry):
            x = gath_vs[lg][pl.ds(q * 16, 16)]
            return tuple(_insert8(list(carry), x))
        top = lax.fori_loop(
            0, KK * S, fin_body,
            tuple(jnp.full((16,), NEG_INF, jnp.float32) for _ in range(KK)))
        for k in range(KK):
            out_v[k, pl.ds(col, 16)] = top[k]

    pltpu.sync_copy(out_v, out_hbm.at[b, cb])


def _topk_sc(xlina, xlinb, gma, gmb):
    mesh = plsc.VectorSubcoreMesh(
        core_axis_name="c", subcore_axis_name="s", num_cores=2,
        num_subcores=16)
    f = pl.kernel(
        _topk_sc_body,
        out_type=jax.ShapeDtypeStruct((B, NCB, KK, CB), jnp.float32),
        mesh=mesh,
        compiler_params=pltpu.CompilerParams(needs_layout_passes=False),
        scratch_types=[
            pltpu.VMEM((G, CB), jnp.float32),
            pltpu.VMEM((G * 16,), jnp.float32),
            pltpu.VMEM((G * 16,), jnp.int32),
            pltpu.VMEM((KK, CB), jnp.float32),
        ] + [pltpu.VMEM((KK * S * 16,), jnp.int32) for _ in range(NLG)]
          + [pltpu.VMEM((KK * S * 16,), jnp.float32) for _ in range(NLG)]
          + [pltpu.SemaphoreType.DMA],
    )
    return f(xlina, xlinb, gma, gmb)


@jax.jit
def kernel(top_k):
    gma, xlina = _tc_pass(top_k)
    gmb = _tc_gm_rest(top_k)
    xlinb = top_k[BT:].reshape(-1)
    out_blk = _topk_sc(xlina, xlinb, gma, gmb)
    return out_blk.transpose(0, 2, 1, 3).reshape(B, KK, C)


# speedup vs baseline: 1.0838x; 1.0009x over previous
import jax
import jax.numpy as jnp
from jax import lax
from jax.experimental import pallas as pl
from jax.experimental.pallas import tpu as pltpu
from jax.experimental.pallas import tpu_sc as plsc

B, T, C = 4, 8192, 1024
KK = 8
S = 16
G = T // S
NCB = 8
CB = C // NCB
NLG = CB // 16
NCG = 32
CGS = G // NCG
RB = 256
BT = 1
BS = B - BT
NEG_INF = float("-inf")


def _tc_body(x_ref, gm_ref, xlin_ref):
    tt = pl.program_id(2)
    for cb in range(NCB):
        slab = x_ref[0, :, pl.ds(cb * CB, CB)]

        @pl.when(tt == 0)
        def _():
            gm_ref[0, cb] = slab

        @pl.when(tt != 0)
        def _():
            gm_ref[0, cb] = jnp.maximum(gm_ref[0, cb], slab)
    for r in range(RB):
        xlin_ref[pl.ds(r * C, C)] = x_ref[0, r, :]


def _tc_pass(x):
    return pl.pallas_call(
        _tc_body,
        grid=(BT, 2, S),
        in_specs=[pl.BlockSpec((1, RB, C), lambda b, h, tt: (b, 2 * tt + h, 0))],
        out_specs=[
            pl.BlockSpec((1, NCB, RB, CB), lambda b, h, tt: (b, 0, h, 0)),
            pl.BlockSpec((RB * C,), lambda b, h, tt: (b * 2 * S + tt * 2 + h)),
        ],
        out_shape=[
            jax.ShapeDtypeStruct((BT, NCB, G, CB), jnp.float32),
            jax.ShapeDtypeStruct((BT * T * C,), jnp.float32),
        ],
        compiler_params=pltpu.CompilerParams(
            dimension_semantics=("arbitrary", "arbitrary", "arbitrary")),
    )(x)


def _gm_body(x_ref, gm_ref):
    acc = x_ref[0, pl.ds(0, G), :]
    for m in range(1, S):
        acc = jnp.maximum(acc, x_ref[0, pl.ds(m * G, G), :])
    gm_ref[0, 0] = acc


def _tc_gm_rest(x):
    return pl.pallas_call(
        _gm_body,
        grid=(BS, NCB),
        in_specs=[pl.BlockSpec((1, T, CB), lambda b, cb: (b + BT, 0, cb))],
        out_specs=pl.BlockSpec((1, 1, G, CB), lambda b, cb: (b, cb, 0, 0)),
        out_shape=jax.ShapeDtypeStruct((BS, NCB, G, CB), jnp.float32),
    )(x)


def _insert8(vals, x):
    out = []
    for i in range(KK):
        c = x > vals[i]
        out.append(jnp.where(c, x, vals[i]))
        x = jnp.where(c, vals[i], x)
    return out


def _insert8_kv(vals, idxs, x, g):
    out_v, out_i = [], []
    for i in range(KK):
        c = x > vals[i]
        out_v.append(jnp.where(c, x, vals[i]))
        out_i.append(jnp.where(c, g, idxs[i]))
        x = jnp.where(c, vals[i], x)
        g = jnp.where(c, idxs[i], g)
    return out_v, out_i


def _topk_sc_body(xlina_hbm, xlinb_hbm, gma_hbm, gmb_hbm, out_hbm,
                  gm_v, candv, candg, out_v, *rest):
    idx_vs = rest[:NLG]
    gath_vs = rest[NLG:2 * NLG]
    sem = rest[2 * NLG]
    cid = lax.axis_index("c")
    sid = lax.axis_index("s")
    wid = sid * 2 + cid
    b = wid // NCB
    cb = lax.rem(wid, NCB)
    is_a = b < BT

    @pl.when(is_a)
    def _():
        pltpu.sync_copy(gma_hbm.at[b, cb], gm_v)

    @pl.when(jnp.logical_not(is_a))
    def _():
        pltpu.sync_copy(gmb_hbm.at[b - BT, cb], gm_v)

    lanes = lax.iota(jnp.int32, 16)
    for lg in range(NLG):
        col = lg * 16

        def coarse_body(cg, carry):
            def row_body(j, acc):
                return jnp.maximum(acc, gm_v[cg * CGS + j, pl.ds(col, 16)])
            m = lax.fori_loop(0, CGS, row_body,
                              jnp.full((16,), NEG_INF, jnp.float32))
            return tuple(_insert8(list(carry), m))
        top0 = lax.fori_loop(
            0, NCG, coarse_body,
            tuple(jnp.full((16,), NEG_INF, jnp.float32) for _ in range(KK)))
        tau0 = top0[KK - 1]

        def scan_body(r, cnt):
            x = gm_v[r, pl.ds(col, 16)]
            msk = x >= tau0
            slot = cnt * 16 + lanes
            plsc.store_scatter(candv, [slot], x, mask=msk)
            plsc.store_scatter(candg, [slot],
                               jnp.full((16,), r, jnp.int32), mask=msk)
            return cnt + msk.astype(jnp.int32)
        cnt = lax.fori_loop(0, G, scan_body, jnp.zeros((16,), jnp.int32))
        maxcnt = jnp.max(cnt)

        def ins_body(r, carry):
            vals = list(carry[:KK])
            idxs = list(carry[KK:])
            valid = r < cnt
            x = jnp.where(valid, candv[pl.ds(r * 16, 16)], NEG_INF)
            g = candg[pl.ds(r * 16, 16)]
            vals, idxs = _insert8_kv(vals, idxs, x, g)
            return tuple(vals) + tuple(idxs)
        init = (tuple(jnp.full((16,), NEG_INF, jnp.float32) for _ in range(KK))
                + tuple(jnp.zeros((16,), jnp.int32) for _ in range(KK)))
        res = lax.fori_loop(0, maxcnt, ins_body, init)
        gids = res[KK:]

        bq = jnp.where(is_a, b, b - BT)
        cbase = bq * (T * C) + cb * CB + col + lanes
        for j in range(KK):
            base = gids[j] * C + cbase
            for m in range(S):
                idx_vs[lg][pl.ds((j * S + m) * 16, 16)] = base + m * (G * C)

        @pl.when(is_a)
        def _():
            pltpu.async_copy(xlina_hbm.at[idx_vs[lg]], gath_vs[lg], sem)

        @pl.when(jnp.logical_not(is_a))
        def _():
            pltpu.async_copy(xlinb_hbm.at[idx_vs[lg]], gath_vs[lg], sem)

    for lg in range(NLG):
        pltpu.make_async_copy(
            xlina_hbm.at[idx_vs[lg]], gath_vs[lg], sem).wait()

    for lg in range(NLG):
        col = lg * 16

        def fin_body(q, carry):
            x = gath_vs[lg][pl.ds(q * 16, 16)]
            return tuple(_insert8(list(carry), x))
        top = lax.fori_loop(
            0, KK * S, fin_body,
            tuple(jnp.full((16,), NEG_INF, jnp.float32) for _ in range(KK)))
        for k in range(KK):
            out_v[k, pl.ds(col, 16)] = top[k]

    pltpu.sync_copy(out_v, out_hbm.at[b, cb])


def _topk_sc(xlina, xlinb, gma, gmb):
    mesh = plsc.VectorSubcoreMesh(
        core_axis_name="c", subcore_axis_name="s", num_cores=2,
        num_subcores=16)
    f = pl.kernel(
        _topk_sc_body,
        out_type=jax.ShapeDtypeStruct((B, NCB, KK, CB), jnp.float32),
        mesh=mesh,
        compiler_params=pltpu.CompilerParams(needs_layout_passes=False),
        scratch_types=[
            pltpu.VMEM((G, CB), jnp.float32),
            pltpu.VMEM((G * 16,), jnp.float32),
            pltpu.VMEM((G * 16,), jnp.int32),
            pltpu.VMEM((KK, CB), jnp.float32),
        ] + [pltpu.VMEM((KK * S * 16,), jnp.int32) for _ in range(NLG)]
          + [pltpu.VMEM((KK * S * 16,), jnp.float32) for _ in range(NLG)]
          + [pltpu.SemaphoreType.DMA],
    )
    return f(xlina, xlinb, gma, gmb)


@jax.jit
def kernel(top_k):
    xlinb = top_k[BT:].reshape(-1)
    gma, xlina = _tc_pass(top_k)
    gmb = _tc_gm_rest(top_k)
    out_blk = _topk_sc(xlina, xlinb, gma, gmb)
    return out_blk.transpose(0, 2, 1, 3).reshape(B, KK, C)


# speedup vs baseline: 1.3085x; 1.2074x over previous
import jax
import jax.numpy as jnp
from jax import lax
from jax.experimental import pallas as pl
from jax.experimental.pallas import tpu as pltpu
from jax.experimental.pallas import tpu_sc as plsc

B, T, C = 4, 8192, 1024
KK = 8
S = 16
G = T // S
NCB = 8
CB = C // NCB
NLG = CB // 16
NCG = 32
CGS = G // NCG
NEG_INF = float("-inf")


def _groupmax_body(x_ref, gm_ref, gmc_ref):
    acc = x_ref[0, pl.ds(0, G), :]
    for m in range(1, S):
        acc = jnp.maximum(acc, x_ref[0, pl.ds(m * G, G), :])
    gm_ref[0, 0] = acc
    cacc = acc[0:NCG, :]
    for j in range(1, CGS):
        cacc = jnp.maximum(cacc, acc[j * NCG:(j + 1) * NCG, :])
    gmc_ref[0, 0] = cacc


def _group_max(x):
    return pl.pallas_call(
        _groupmax_body,
        grid=(B, NCB),
        in_specs=[pl.BlockSpec((1, T, CB), lambda b, cb: (b, 0, cb))],
        out_specs=[
            pl.BlockSpec((1, 1, G, CB), lambda b, cb: (b, cb, 0, 0)),
            pl.BlockSpec((1, 1, NCG, CB), lambda b, cb: (b, cb, 0, 0)),
        ],
        out_shape=[
            jax.ShapeDtypeStruct((B, NCB, G, CB), jnp.float32),
            jax.ShapeDtypeStruct((B, NCB, NCG, CB), jnp.float32),
        ],
    )(x)


def _insert8(vals, x):
    out = []
    for i in range(KK):
        c = x > vals[i]
        out.append(jnp.where(c, x, vals[i]))
        x = jnp.where(c, vals[i], x)
    return out


def _insert8_kv(vals, idxs, x, g):
    out_v, out_i = [], []
    for i in range(KK):
        c = x > vals[i]
        out_v.append(jnp.where(c, x, vals[i]))
        out_i.append(jnp.where(c, g, idxs[i]))
        x = jnp.where(c, vals[i], x)
        g = jnp.where(c, idxs[i], g)
    return out_v, out_i


def _topk_sc_body(xflat_hbm, gm_hbm, gmc_hbm, out_hbm,
                  gm_v, gmc_v, candv, candg, out_v, *rest):
    idx_vs = rest[:NLG]
    gath_vs = rest[NLG:2 * NLG]
    sem = rest[2 * NLG]
    cid = lax.axis_index("c")
    sid = lax.axis_index("s")
    wid = sid * 2 + cid
    b = wid // NCB
    cb = lax.rem(wid, NCB)

    pltpu.sync_copy(gm_hbm.at[b, cb], gm_v)
    pltpu.sync_copy(gmc_hbm.at[b, cb], gmc_v)

    lanes = lax.iota(jnp.int32, 16)
    tau2s = []
    for lg in range(NLG):
        col = lg * 16

        def coarse_body(cg, carry):
            return tuple(_insert8(list(carry), gmc_v[cg, pl.ds(col, 16)]))
        top0 = lax.fori_loop(
            0, NCG, coarse_body,
            tuple(jnp.full((16,), NEG_INF, jnp.float32) for _ in range(KK)))
        tau0 = top0[KK - 1]

        def scan_body(r, cnt):
            x = gm_v[r, pl.ds(col, 16)]
            msk = x >= tau0
            slot = cnt * 16 + lanes
            plsc.store_scatter(candv, [slot], x, mask=msk)
            plsc.store_scatter(candg, [slot],
                               jnp.full((16,), r, jnp.int32), mask=msk)
            return cnt + msk.astype(jnp.int32)
        cnt = lax.fori_loop(0, G, scan_body, jnp.zeros((16,), jnp.int32))
        maxcnt = jnp.max(cnt)

        def ins_body(r, carry):
            vals = list(carry[:KK])
            idxs = list(carry[KK:])
            valid = r < cnt
            x = jnp.where(valid, candv[pl.ds(r * 16, 16)], NEG_INF)
            g = candg[pl.ds(r * 16, 16)]
            vals, idxs = _insert8_kv(vals, idxs, x, g)
            return tuple(vals) + tuple(idxs)
        init = (tuple(jnp.full((16,), NEG_INF, jnp.float32) for _ in range(KK))
                + tuple(jnp.zeros((16,), jnp.int32) for _ in range(KK)))
        res = lax.fori_loop(0, maxcnt, ins_body, init)
        gids = res[KK:]
        tau2s.append(res[KK - 1])

        cbase = b * (T * C) + cb * CB + col + lanes
        for j in range(KK):
            base = gids[j] * C + cbase
            for m in range(S):
                idx_vs[lg][pl.ds((j * S + m) * 16, 16)] = base + m * (G * C)
        pltpu.async_copy(xflat_hbm.at[idx_vs[lg]], gath_vs[lg], sem)

    for lg in range(NLG):
        pltpu.make_async_copy(
            xflat_hbm.at[idx_vs[lg]], gath_vs[lg], sem).wait()

    for lg in range(NLG):
        col = lg * 16
        tau2 = tau2s[lg]

        def fscan_body(q, cnt):
            x = gath_vs[lg][pl.ds(q * 16, 16)]
            msk = x >= tau2
            plsc.store_scatter(candv, [cnt * 16 + lanes], x, mask=msk)
            return cnt + msk.astype(jnp.int32)
        cnt2 = lax.fori_loop(0, KK * S, fscan_body,
                             jnp.zeros((16,), jnp.int32))
        maxcnt2 = jnp.max(cnt2)

        def fin_body(r, carry):
            valid = r < cnt2
            x = jnp.where(valid, candv[pl.ds(r * 16, 16)], NEG_INF)
            return tuple(_insert8(list(carry), x))
        top = lax.fori_loop(
            0, maxcnt2, fin_body,
            tuple(jnp.full((16,), NEG_INF, jnp.float32) for _ in range(KK)))
        for k in range(KK):
            out_v[k, pl.ds(col, 16)] = top[k]

    pltpu.sync_copy(out_v, out_hbm.at[b, cb])


def _topk_sc(xflat, gm, gmc):
    mesh = plsc.VectorSubcoreMesh(
        core_axis_name="c", subcore_axis_name="s", num_cores=2,
        num_subcores=16)
    f = pl.kernel(
        _topk_sc_body,
        out_type=jax.ShapeDtypeStruct((B, NCB, KK, CB), jnp.float32),
        mesh=mesh,
        compiler_params=pltpu.CompilerParams(needs_layout_passes=False),
        scratch_types=[
            pltpu.VMEM((G, CB), jnp.float32),
            pltpu.VMEM((NCG, CB), jnp.float32),
            pltpu.VMEM((G * 16,), jnp.float32),
            pltpu.VMEM((G * 16,), jnp.int32),
            pltpu.VMEM((KK, CB), jnp.float32),
        ] + [pltpu.VMEM((KK * S * 16,), jnp.int32) for _ in range(NLG)]
          + [pltpu.VMEM((KK * S * 16,), jnp.float32) for _ in range(NLG)]
          + [pltpu.SemaphoreType.DMA],
    )
    return f(xflat, gm, gmc)


@jax.jit
def kernel(top_k):
    gm, gmc = _group_max(top_k)
    out_blk = _topk_sc(top_k.reshape(-1), gm, gmc)
    return out_blk.transpose(0, 2, 1, 3).reshape(B, KK, C)


# speedup vs baseline: 1.3411x; 1.0249x over previous
import jax
import jax.numpy as jnp
from jax import lax
from jax.experimental import pallas as pl
from jax.experimental.pallas import tpu as pltpu
from jax.experimental.pallas import tpu_sc as plsc

B, T, C = 4, 8192, 1024
KK = 8
S = 16
G = T // S
NCB = 8
CB = C // NCB
NLG = CB // 16
NCG = 32
CGS = G // NCG
NEG_INF = float("-inf")


def _groupmax_body(x_ref, gm_ref, gmc_ref):
    acc = x_ref[0, pl.ds(0, G), :]
    for m in range(1, S):
        acc = jnp.maximum(acc, x_ref[0, pl.ds(m * G, G), :])
    gm_ref[0, 0] = acc
    cacc = acc[0:NCG, :]
    for j in range(1, CGS):
        cacc = jnp.maximum(cacc, acc[j * NCG:(j + 1) * NCG, :])
    gmc_ref[0, 0] = cacc


def _group_max(x):
    return pl.pallas_call(
        _groupmax_body,
        grid=(B, NCB),
        in_specs=[pl.BlockSpec((1, T, CB), lambda b, cb: (b, 0, cb))],
        out_specs=[
            pl.BlockSpec((1, 1, G, CB), lambda b, cb: (b, cb, 0, 0)),
            pl.BlockSpec((1, 1, NCG, CB), lambda b, cb: (b, cb, 0, 0)),
        ],
        out_shape=[
            jax.ShapeDtypeStruct((B, NCB, G, CB), jnp.float32),
            jax.ShapeDtypeStruct((B, NCB, NCG, CB), jnp.float32),
        ],
    )(x)


def _insert8(vals, x):
    out = []
    for i in range(KK):
        c = x > vals[i]
        out.append(jnp.where(c, x, vals[i]))
        x = jnp.where(c, vals[i], x)
    return out


def _insert8_kv(vals, idxs, x, g):
    out_v, out_i = [], []
    for i in range(KK):
        c = x > vals[i]
        out_v.append(jnp.where(c, x, vals[i]))
        out_i.append(jnp.where(c, g, idxs[i]))
        x = jnp.where(c, vals[i], x)
        g = jnp.where(c, idxs[i], g)
    return out_v, out_i


def _topk_sc_body(xflat_hbm, gm_hbm, gmc_hbm, out_hbm,
                  gm_v, gmc_v, candv, candg, out_v, *rest):
    idx_vs = rest[:NLG]
    gath_vs = rest[NLG:2 * NLG]
    sem = rest[2 * NLG]
    cid = lax.axis_index("c")
    sid = lax.axis_index("s")
    wid = sid * 2 + cid
    b = wid // NCB
    cb = lax.rem(wid, NCB)

    pltpu.sync_copy(gm_hbm.at[b, cb], gm_v)
    pltpu.sync_copy(gmc_hbm.at[b, cb], gmc_v)

    lanes = lax.iota(jnp.int32, 16)
    for lg in range(NLG):
        col = lg * 16

        def coarse_body(cg, carry):
            return tuple(_insert8(list(carry), gmc_v[cg, pl.ds(col, 16)]))
        top0 = lax.fori_loop(
            0, NCG, coarse_body,
            tuple(jnp.full((16,), NEG_INF, jnp.float32) for _ in range(KK)))
        tau0 = top0[KK - 1]

        def scan_body(r, cnt):
            x = gm_v[r, pl.ds(col, 16)]
            msk = x >= tau0
            slot = cnt * 16 + lanes
            plsc.store_scatter(candv, [slot], x, mask=msk)
            plsc.store_scatter(candg, [slot],
                               jnp.full((16,), r, jnp.int32), mask=msk)
            return cnt + msk.astype(jnp.int32)
        cnt = lax.fori_loop(0, G, scan_body, jnp.zeros((16,), jnp.int32))
        maxcnt = jnp.max(cnt)

        def ins_body(r, carry):
            vals = list(carry[:KK])
            idxs = list(carry[KK:])
            valid = r < cnt
            x = jnp.where(valid, candv[pl.ds(r * 16, 16)], NEG_INF)
            g = candg[pl.ds(r * 16, 16)]
            vals, idxs = _insert8_kv(vals, idxs, x, g)
            return tuple(vals) + tuple(idxs)
        init = (tuple(jnp.full((16,), NEG_INF, jnp.float32) for _ in range(KK))
                + tuple(jnp.zeros((16,), jnp.int32) for _ in range(KK)))
        res = lax.fori_loop(0, maxcnt, ins_body, init)
        gids = res[KK:]

        cbase = b * (T * C) + cb * CB + col + lanes
        for j in range(KK):
            base = gids[j] * C + cbase
            for m in range(S):
                idx_vs[lg][pl.ds((j * S + m) * 16, 16)] = base + m * (G * C)
        pltpu.async_copy(xflat_hbm.at[idx_vs[lg]], gath_vs[lg], sem)

    for lg in range(NLG):
        pltpu.make_async_copy(
            xflat_hbm.at[idx_vs[lg]], gath_vs[lg], sem).wait()

    for lg in range(NLG):
        col = lg * 16

        def fin_body(q, carry):
            x = gath_vs[lg][pl.ds(q * 16, 16)]
            return tuple(_insert8(list(carry), x))
        top = lax.fori_loop(
            0, KK * S, fin_body,
            tuple(jnp.full((16,), NEG_INF, jnp.float32) for _ in range(KK)))
        for k in range(KK):
            out_v[k, pl.ds(col, 16)] = top[k]

    pltpu.sync_copy(out_v, out_hbm.at[b, :, pl.ds(cb * CB, CB)])


def _topk_sc(xflat, gm, gmc):
    mesh = plsc.VectorSubcoreMesh(
        core_axis_name="c", subcore_axis_name="s", num_cores=2,
        num_subcores=16)
    f = pl.kernel(
        _topk_sc_body,
        out_type=jax.ShapeDtypeStruct((B, KK, C), jnp.float32),
        mesh=mesh,
        compiler_params=pltpu.CompilerParams(needs_layout_passes=False),
        scratch_types=[
            pltpu.VMEM((G, CB), jnp.float32),
            pltpu.VMEM((NCG, CB), jnp.float32),
            pltpu.VMEM((G * 16,), jnp.float32),
            pltpu.VMEM((G * 16,), jnp.int32),
            pltpu.VMEM((KK, CB), jnp.float32),
        ] + [pltpu.VMEM((KK * S * 16,), jnp.int32) for _ in range(NLG)]
          + [pltpu.VMEM((KK * S * 16,), jnp.float32) for _ in range(NLG)]
          + [pltpu.SemaphoreType.DMA],
    )
    return f(xflat, gm, gmc)


@jax.jit
def kernel(top_k):
    gm, gmc = _group_max(top_k)
    return _topk_sc(top_k.reshape(-1), gm, gmc)
